# baseline (speedup 1.0000x reference)
import jax
import jax.numpy as jnp
from jax.experimental import pallas as pl
from jax.experimental.pallas import tpu as pltpu

_PATCH = 14
_NPOS = _PATCH * _PATCH
_NLEV = 256
_IB = 64


def _encoder_body(x_ref, pw_ref, lw_ref, out_ref):
    pw = pw_ref[...]
    lw = lw_ref[...]
    iota = jax.lax.broadcasted_iota(jnp.int32, (_NPOS, _NLEV), 1)
    ones_row = jnp.ones((1, _NPOS), jnp.float32)
    for i in range(_IB):
        idx = jnp.round(x_ref[i] * (_NLEV - 1.0)).astype(jnp.int32)
        cnt = (idx[0][:, None] == iota).astype(jnp.float32)
        for p in range(1, 4):
            cnt += (idx[p][:, None] == iota).astype(jnp.float32)
        m = jax.lax.dot_general(
            cnt.astype(jnp.bfloat16), lw, (((1,), (0,)), ((), ())),
            preferred_element_type=jnp.float32,
        )
        s = jax.lax.dot_general(
            ones_row, m * pw, (((1,), (0,)), ((), ())),
            preferred_element_type=jnp.float32,
        )
        out_ref[i, :] = jnp.where(s[0] > 0.0, 1.0, -1.0)


def kernel(x, position_weight, level_weight):
    B, C, H, W = x.shape
    p = _PATCH
    D = position_weight.shape[1]
    x_pj = x.reshape(B, C, H // p, p, W // p, p)
    x_pj = x_pj.transpose(0, 1, 2, 4, 3, 5).reshape(B, 4, p * p)
    lw_bf16 = level_weight.astype(jnp.bfloat16)

    grid = (B // _IB,)
    return pl.pallas_call(
        _encoder_body,
        grid=grid,
        in_specs=[
            pl.BlockSpec((_IB, 4, _NPOS), lambda i: (i, 0, 0)),
            pl.BlockSpec((_NPOS, D), lambda i: (0, 0)),
            pl.BlockSpec((_NLEV, D), lambda i: (0, 0)),
        ],
        out_specs=pl.BlockSpec((_IB, D), lambda i: (i, 0)),
        out_shape=jax.ShapeDtypeStruct((B, D), jnp.float32),
    )(x_pj, position_weight, lw_bf16)
